# baseline (speedup 1.0000x reference)
import jax
import jax.numpy as jnp
from jax import lax
from jax.experimental import pallas as pl
from jax.experimental.pallas import tpu as pltpu
from jax.experimental.pallas import tpu_sc as plsc

N = 10000
E = 320000
D = 128
H = 8
C1, C2 = 16, 8
HC1, HC2 = H * C1, H * C2
ROW1, ROW2 = HC1 + 16, HC2 + 16
NPAD = 10240
NC, NS = 2, 16
K = 96
PT = 10368
NCH = PT // K
ET = E + N
ET_PAD = NC * NS * PT
RPT = NPAD // NS
BR = 1000


def _make_sc_gat(hc, row):
    jb = hc // 16

    def body(src_hbm, dst_hbm, alpha_hbm, h_hbm, m_hbm, out_hbm,
             table, srcv, dstv, av, bv, hbuf, msgc, wbuf, mbuf,
             sem_a, sem_b, sem_h):
        cid = lax.axis_index("c")
        sid = lax.axis_index("s")
        lane = lax.iota(jnp.int32, 16)
        half = lane // 8
        hcol = lane - 8 * half
        zero16 = jnp.zeros((16,), jnp.float32)

        pltpu.sync_copy(m_hbm, mbuf)

        def zbody(i, c):
            iv = jnp.full((16,), i, jnp.int32)
            for j in range(row // 16):
                plsc.store_scatter(msgc, [iv, lane + 16 * j], zero16)
            return c
        lax.fori_loop(0, K, zbody, 0, unroll=4)
        nfull = RPT // K
        for b in range(nfull):
            pltpu.sync_copy(msgc, table.at[pl.ds(sid * RPT + b * K, K)])
        rem = RPT - nfull * K
        if rem:
            pltpu.sync_copy(msgc.at[pl.ds(0, rem)],
                            table.at[pl.ds(sid * RPT + nfull * K, rem)])
        plsc.subcore_barrier()

        def chunk(g, c):
            base = (cid * NS + sid) * PT + g * K
            pltpu.sync_copy(src_hbm.at[pl.ds(base, K)], srcv)
            pltpu.sync_copy(dst_hbm.at[pl.ds(base, K)], dstv)
            ca = pltpu.async_copy(alpha_hbm.at[srcv], av, sem_a)
            cb = pltpu.async_copy(alpha_hbm.at[dstv], bv, sem_b)
            ch = pltpu.async_copy(h_hbm.at[srcv], hbuf, sem_h)
            ca.wait()
            cb.wait()
            ch.wait()
            mv = mbuf[...]

            def wbody(v, cc):
                rows = 2 * v + half
                xs = plsc.load_gather(av, [rows, hcol])
                xd = plsc.load_gather(bv, [rows, hcol + 8])
                e = xs + xd
                e = jnp.where(e >= 0.0, e, 0.2 * e)
                w = jnp.exp(e - mv)
                wbuf[pl.ds(v * 16, 16)] = w
                plsc.store_scatter(msgc, [rows, hcol + hc], w)
                plsc.store_scatter(msgc, [rows, hcol + hc + 8], w)
                return cc
            lax.fori_loop(0, K // 2, wbody, 0, unroll=4)

            def mbody(e2, cc):
                ev = jnp.full((16,), e2, jnp.int32)
                for j in range(jb):
                    if hc == 128:
                        widx = jnp.full((16,), e2 * 8 + j, jnp.int32)
                    else:
                        widx = 8 * e2 + 2 * j + half
                    ws = plsc.load_gather(wbuf, [widx])
                    hv = plsc.load_gather(hbuf, [ev, lane + 16 * j])
                    plsc.store_scatter(msgc, [ev, lane + 16 * j], hv * ws)
                return cc
            lax.fori_loop(0, K, mbody, 0, unroll=2)

            pltpu.sync_copy(msgc, table.at[dstv], add=True)
            return c
        lax.fori_loop(0, NCH, chunk, 0)

        plsc.subcore_barrier()
        for b in range(2):
            r0 = sid * RPT + b * (RPT // 2)
            pltpu.sync_copy(table.at[pl.ds(r0, RPT // 2)],
                            out_hbm.at[cid, pl.ds(r0, RPT // 2)])

    return body


def _sc_call(hc, row):
    mesh = plsc.VectorSubcoreMesh(
        core_axis_name="c", subcore_axis_name="s", num_cores=NC, num_subcores=NS)
    return pl.kernel(
        _make_sc_gat(hc, row),
        out_type=jax.ShapeDtypeStruct((NC, NPAD, row), jnp.float32),
        mesh=mesh,
        compiler_params=pltpu.CompilerParams(
            needs_layout_passes=False, use_tc_tiling_on_sc=False),
        scratch_types=[
            pltpu.VMEM_SHARED((NPAD, row), jnp.float32),
            pltpu.VMEM((K,), jnp.int32),
            pltpu.VMEM((K,), jnp.int32),
            pltpu.VMEM((K, 16), jnp.float32),
            pltpu.VMEM((K, 16), jnp.float32),
            pltpu.VMEM((K, hc), jnp.float32),
            pltpu.VMEM((K, row), jnp.float32),
            pltpu.VMEM((K * 8,), jnp.float32),
            pltpu.VMEM((16,), jnp.float32),
            pltpu.SemaphoreType.DMA,
            pltpu.SemaphoreType.DMA,
            pltpu.SemaphoreType.DMA,
        ],
    )


_sc_gat1 = _sc_call(HC1, ROW1)
_sc_gat2 = _sc_call(HC2, ROW2)


def _tc_embed(x_ref, w_ref, asel_ref, h_ref, a_ref, m_ref):
    i = pl.program_id(0)
    hb = jnp.dot(x_ref[...], w_ref[...], preferred_element_type=jnp.float32)
    ab = jnp.dot(hb, asel_ref[...], preferred_element_type=jnp.float32)
    h_ref[...] = hb
    a_ref[...] = ab
    bm = jnp.max(ab, axis=0, keepdims=True)

    @pl.when(i == 0)
    def _init():
        m_ref[...] = bm

    @pl.when(i != 0)
    def _acc():
        m_ref[...] = jnp.maximum(m_ref[...], bm)


def _tc_mid(p0_ref, p1_ref, rsel_ref, b1_ref, w2_ref, asel_ref,
            h_ref, a_ref, m_ref):
    i = pl.program_id(0)
    p = p0_ref[0] + p1_ref[0]
    num = p[:, :HC1]
    den = p[:, HC1:HC1 + 8]
    rec = 1.0 / (den + 1e-30)
    o1 = num * jnp.dot(rec, rsel_ref[...], preferred_element_type=jnp.float32)
    o1 = jnp.maximum(o1 + b1_ref[...], 0.0)
    h2 = jnp.dot(o1, w2_ref[...], preferred_element_type=jnp.float32)
    a2 = jnp.dot(h2, asel_ref[...], preferred_element_type=jnp.float32)
    h_ref[...] = h2
    a_ref[...] = a2
    bm = jnp.max(a2, axis=0, keepdims=True)

    @pl.when(i == 0)
    def _init():
        m_ref[...] = bm

    @pl.when(i != 0)
    def _acc():
        m_ref[...] = jnp.maximum(m_ref[...], bm)


def _tc_out(p0_ref, p1_ref, rsel_ref, b2_ref, o_ref):
    p = p0_ref[0] + p1_ref[0]
    num = p[:, :HC2]
    den = p[:, HC2:HC2 + 8]
    rec = 1.0 / (den + 1e-30)
    o = num * jnp.dot(rec, rsel_ref[...], preferred_element_type=jnp.float32)
    o = o + b2_ref[...]
    m = jnp.max(o, axis=1, keepdims=True)
    l = o - m
    o_ref[...] = l - jnp.log(jnp.sum(jnp.exp(l), axis=1, keepdims=True))


_embed_call = pl.pallas_call(
    _tc_embed,
    grid=(N // BR,),
    in_specs=[
        pl.BlockSpec((BR, D), lambda i: (i, 0)),
        pl.BlockSpec((D, D), lambda i: (0, 0)),
        pl.BlockSpec((D, 16), lambda i: (0, 0)),
    ],
    out_specs=[
        pl.BlockSpec((BR, D), lambda i: (i, 0)),
        pl.BlockSpec((BR, 16), lambda i: (i, 0)),
        pl.BlockSpec((1, 16), lambda i: (0, 0)),
    ],
    out_shape=[
        jax.ShapeDtypeStruct((N, D), jnp.float32),
        jax.ShapeDtypeStruct((N, 16), jnp.float32),
        jax.ShapeDtypeStruct((1, 16), jnp.float32),
    ],
)

_mid_call = pl.pallas_call(
    _tc_mid,
    grid=(N // BR,),
    in_specs=[
        pl.BlockSpec((1, BR, ROW1), lambda i: (0, i, 0)),
        pl.BlockSpec((1, BR, ROW1), lambda i: (1, i, 0)),
        pl.BlockSpec((H, HC1), lambda i: (0, 0)),
        pl.BlockSpec((1, HC1), lambda i: (0, 0)),
        pl.BlockSpec((HC1, HC2), lambda i: (0, 0)),
        pl.BlockSpec((HC2, 16), lambda i: (0, 0)),
    ],
    out_specs=[
        pl.BlockSpec((BR, HC2), lambda i: (i, 0)),
        pl.BlockSpec((BR, 16), lambda i: (i, 0)),
        pl.BlockSpec((1, 16), lambda i: (0, 0)),
    ],
    out_shape=[
        jax.ShapeDtypeStruct((N, HC2), jnp.float32),
        jax.ShapeDtypeStruct((N, 16), jnp.float32),
        jax.ShapeDtypeStruct((1, 16), jnp.float32),
    ],
)

_out_call = pl.pallas_call(
    _tc_out,
    grid=(N // BR,),
    in_specs=[
        pl.BlockSpec((1, BR, ROW2), lambda i: (0, i, 0)),
        pl.BlockSpec((1, BR, ROW2), lambda i: (1, i, 0)),
        pl.BlockSpec((H, HC2), lambda i: (0, 0)),
        pl.BlockSpec((1, HC2), lambda i: (0, 0)),
    ],
    out_specs=pl.BlockSpec((BR, HC2), lambda i: (i, 0)),
    out_shape=jax.ShapeDtypeStruct((N, HC2), jnp.float32),
)


def _asel(a_s, a_d, c):
    eye = jnp.eye(H, dtype=jnp.float32)
    s = (a_s[:, :, None] * eye[:, None, :]).reshape(H * c, H)
    d = (a_d[:, :, None] * eye[:, None, :]).reshape(H * c, H)
    return jnp.concatenate([s, d], axis=1)


def _mtile(m):
    mm = m[0, :8] + m[0, 8:]
    mm = jnp.where(mm >= 0.0, mm, 0.2 * mm)
    return jnp.concatenate([mm, mm])


def kernel(x, edge_index, W1, a1_src, a1_dst, b1, W2, a2_src, a2_dst, b2):
    loop = jnp.arange(N, dtype=jnp.int32)
    npad = ET_PAD - ET
    src = jnp.concatenate([edge_index[0], loop, jnp.zeros((npad,), jnp.int32)])
    dst = jnp.concatenate([edge_index[1], loop, jnp.full((npad,), N, jnp.int32)])

    asel1 = _asel(a1_src, a1_dst, C1)
    asel2 = _asel(a2_src, a2_dst, C2)
    rsel1 = jnp.repeat(jnp.eye(H, dtype=jnp.float32), C1, axis=1)
    rsel2 = jnp.repeat(jnp.eye(H, dtype=jnp.float32), C2, axis=1)

    h1, al1, m1 = _embed_call(x, W1, asel1)
    part1 = _sc_gat1(src, dst, al1, h1, _mtile(m1))
    h2, al2, m2 = _mid_call(part1, part1, rsel1, b1.reshape(1, HC1), W2, asel2)
    part2 = _sc_gat2(src, dst, al2, h2, _mtile(m2))
    return _out_call(part2, part2, rsel2, b2.reshape(1, HC2))


# speedup vs baseline: 1.5471x; 1.5471x over previous
import jax
import jax.numpy as jnp
from jax import lax
from jax.experimental import pallas as pl
from jax.experimental.pallas import tpu as pltpu
from jax.experimental.pallas import tpu_sc as plsc

N = 10000
E = 320000
D = 128
H = 8
C1, C2 = 16, 8
HC1, HC2 = H * C1, H * C2
ROW1, ROW2 = HC1 + 16, HC2 + 16
NPAD = 10016
NC, NS = 2, 16
K = 64
PT = 10368
NCH = PT // K
ET = E + N
ET_PAD = NC * NS * PT
RPT = NPAD // NS
BR = 1000


def _make_sc_gat(hc, row):
    jb = hc // 16

    def body(src_hbm, dst_hbm, alpha_hbm, h_hbm, m_hbm, out_hbm,
             table, srcv0, srcv1, dstv0, dstv1, sdst0, sdst1, dprime,
             av0, av1, bv0, bv1, hb0, hb1, mg0, mg1, wbuf, mbuf,
             gsem0, gsem1, isem0, isem1, ssem0, ssem1):
        srcv, dstv, sdst = [srcv0, srcv1], [dstv0, dstv1], [sdst0, sdst1]
        av, bv, hbuf, msgc = [av0, av1], [bv0, bv1], [hb0, hb1], [mg0, mg1]
        gsem, isem, ssem = [gsem0, gsem1], [isem0, isem1], [ssem0, ssem1]
        cid = lax.axis_index("c")
        sid = lax.axis_index("s")
        tb = (cid * NS + sid) * PT
        lane = lax.iota(jnp.int32, 16)
        half = lane // 8
        hcol = lane - 8 * half
        zero16 = jnp.zeros((16,), jnp.float32)
        izero16 = jnp.zeros((16,), jnp.int32)

        pltpu.sync_copy(m_hbm, mbuf)

        def zbody(i, c):
            iv = jnp.full((16,), i, jnp.int32)
            for j in range(row // 16):
                plsc.store_scatter(msgc[0], [iv, lane + 16 * j], zero16)
                plsc.store_scatter(msgc[1], [iv, lane + 16 * j], zero16)
            return c
        lax.fori_loop(0, K, zbody, 0, unroll=4)
        for i in range(K // 16):
            dprime[pl.ds(16 * i, 16)] = izero16
        nfull = RPT // K
        for b in range(nfull):
            pltpu.sync_copy(msgc[0], table.at[pl.ds(sid * RPT + b * K, K)])
        rem = RPT - nfull * K
        if rem:
            pltpu.sync_copy(msgc[0].at[pl.ds(0, rem)],
                            table.at[pl.ds(sid * RPT + nfull * K, rem)])
        plsc.subcore_barrier()

        def issue_idx(p, gi):
            base = tb + gi * K
            pltpu.async_copy(src_hbm.at[pl.ds(base, K)], srcv[p], isem[p])
            pltpu.async_copy(dst_hbm.at[pl.ds(base, K)], dstv[p], isem[p])

        def wait_idx(p):
            pltpu.make_async_copy(src_hbm.at[pl.ds(0, K)], srcv[p], isem[p]).wait()
            pltpu.make_async_copy(dst_hbm.at[pl.ds(0, K)], dstv[p], isem[p]).wait()

        def issue_g(p):
            pltpu.async_copy(alpha_hbm.at[srcv[p]], av[p], gsem[p])
            pltpu.async_copy(alpha_hbm.at[dstv[p]], bv[p], gsem[p])
            pltpu.async_copy(h_hbm.at[srcv[p]], hbuf[p], gsem[p])

        def wait_g(p):
            pltpu.make_async_copy(alpha_hbm.at[srcv[p]], av[p], gsem[p]).wait()
            pltpu.make_async_copy(alpha_hbm.at[dstv[p]], bv[p], gsem[p]).wait()
            pltpu.make_async_copy(h_hbm.at[srcv[p]], hbuf[p], gsem[p]).wait()

        def drain_s(p):
            pltpu.make_async_copy(msgc[p], table.at[dprime], ssem[p]).wait()

        pltpu.async_copy(msgc[0], table.at[dprime], ssem[0], add=True)
        pltpu.async_copy(msgc[1], table.at[dprime], ssem[1], add=True)
        issue_idx(0, 0)
        wait_idx(0)
        issue_g(0)
        issue_idx(1, 1)

        def phase(p, g):
            wait_g(p)
            drain_s(p)
            for i in range(K // 16):
                sdst[p][pl.ds(16 * i, 16)] = dstv[p][pl.ds(16 * i, 16)]
            wait_idx(1 - p)
            issue_g(1 - p)
            issue_idx(p, jnp.minimum(g + 2, NCH - 1))
            mv = mbuf[...]

            def wbody(v, cc):
                rows = 2 * v + half
                xs = plsc.load_gather(av[p], [rows, hcol])
                xd = plsc.load_gather(bv[p], [rows, hcol + 8])
                e = xs + xd
                e = jnp.where(e >= 0.0, e, 0.2 * e)
                w = jnp.exp(e - mv)
                wbuf[pl.ds(v * 16, 16)] = w
                plsc.store_scatter(msgc[p], [rows, hcol + hc], w)
                plsc.store_scatter(msgc[p], [rows, hcol + hc + 8], w)
                return cc
            lax.fori_loop(0, K // 2, wbody, 0, unroll=4)

            def mbody(e2, cc):
                ev = jnp.full((16,), e2, jnp.int32)
                for j in range(jb):
                    if hc == 128:
                        widx = jnp.full((16,), e2 * 8 + j, jnp.int32)
                    else:
                        widx = 8 * e2 + 2 * j + half
                    ws = plsc.load_gather(wbuf, [widx])
                    hv = plsc.load_gather(hbuf[p], [ev, lane + 16 * j])
                    plsc.store_scatter(msgc[p], [ev, lane + 16 * j], hv * ws)
                return cc
            lax.fori_loop(0, K, mbody, 0, unroll=2)

            pltpu.async_copy(msgc[p], table.at[sdst[p]], ssem[p], add=True)

        def loop(gg, c):
            phase(0, 2 * gg)
            phase(1, 2 * gg + 1)
            return c
        lax.fori_loop(0, NCH // 2, loop, 0)

        wait_g(0)
        wait_idx(1)
        drain_s(0)
        drain_s(1)

        plsc.subcore_barrier()
        for b in range(2):
            r0 = sid * RPT + b * (RPT // 2)
            pltpu.sync_copy(table.at[pl.ds(r0, RPT // 2)],
                            out_hbm.at[cid, pl.ds(r0, RPT // 2)])

    return body


def _sc_call(hc, row):
    mesh = plsc.VectorSubcoreMesh(
        core_axis_name="c", subcore_axis_name="s", num_cores=NC, num_subcores=NS)
    return pl.kernel(
        _make_sc_gat(hc, row),
        out_type=jax.ShapeDtypeStruct((NC, NPAD, row), jnp.float32),
        mesh=mesh,
        compiler_params=pltpu.CompilerParams(
            needs_layout_passes=False, use_tc_tiling_on_sc=False),
        scratch_types=(
            [pltpu.VMEM_SHARED((NPAD, row), jnp.float32)]
            + [pltpu.VMEM((K,), jnp.int32) for _ in range(7)]
            + [pltpu.VMEM((K, 16), jnp.float32) for _ in range(4)]
            + [pltpu.VMEM((K, hc), jnp.float32) for _ in range(2)]
            + [pltpu.VMEM((K, row), jnp.float32) for _ in range(2)]
            + [pltpu.VMEM((K * 8,), jnp.float32),
               pltpu.VMEM((16,), jnp.float32)]
            + [pltpu.SemaphoreType.DMA for _ in range(6)]
        ),
    )


_sc_gat1 = _sc_call(HC1, ROW1)
_sc_gat2 = _sc_call(HC2, ROW2)


def _tc_embed(x_ref, w_ref, asel_ref, h_ref, a_ref, m_ref):
    i = pl.program_id(0)
    hb = jnp.dot(x_ref[...], w_ref[...], preferred_element_type=jnp.float32)
    ab = jnp.dot(hb, asel_ref[...], preferred_element_type=jnp.float32)
    h_ref[...] = hb
    a_ref[...] = ab
    bm = jnp.max(ab, axis=0, keepdims=True)

    @pl.when(i == 0)
    def _init():
        m_ref[...] = bm

    @pl.when(i != 0)
    def _acc():
        m_ref[...] = jnp.maximum(m_ref[...], bm)


def _tc_mid(p0_ref, p1_ref, rsel_ref, b1_ref, w2_ref, asel_ref,
            h_ref, a_ref, m_ref):
    i = pl.program_id(0)
    p = p0_ref[0] + p1_ref[0]
    num = p[:, :HC1]
    den = p[:, HC1:HC1 + 8]
    rec = 1.0 / (den + 1e-30)
    o1 = num * jnp.dot(rec, rsel_ref[...], preferred_element_type=jnp.float32)
    o1 = jnp.maximum(o1 + b1_ref[...], 0.0)
    h2 = jnp.dot(o1, w2_ref[...], preferred_element_type=jnp.float32)
    a2 = jnp.dot(h2, asel_ref[...], preferred_element_type=jnp.float32)
    h_ref[...] = h2
    a_ref[...] = a2
    bm = jnp.max(a2, axis=0, keepdims=True)

    @pl.when(i == 0)
    def _init():
        m_ref[...] = bm

    @pl.when(i != 0)
    def _acc():
        m_ref[...] = jnp.maximum(m_ref[...], bm)


def _tc_out(p0_ref, p1_ref, rsel_ref, b2_ref, o_ref):
    p = p0_ref[0] + p1_ref[0]
    num = p[:, :HC2]
    den = p[:, HC2:HC2 + 8]
    rec = 1.0 / (den + 1e-30)
    o = num * jnp.dot(rec, rsel_ref[...], preferred_element_type=jnp.float32)
    o = o + b2_ref[...]
    m = jnp.max(o, axis=1, keepdims=True)
    l = o - m
    o_ref[...] = l - jnp.log(jnp.sum(jnp.exp(l), axis=1, keepdims=True))


_embed_call = pl.pallas_call(
    _tc_embed,
    grid=(N // BR,),
    in_specs=[
        pl.BlockSpec((BR, D), lambda i: (i, 0)),
        pl.BlockSpec((D, D), lambda i: (0, 0)),
        pl.BlockSpec((D, 16), lambda i: (0, 0)),
    ],
    out_specs=[
        pl.BlockSpec((BR, D), lambda i: (i, 0)),
        pl.BlockSpec((BR, 16), lambda i: (i, 0)),
        pl.BlockSpec((1, 16), lambda i: (0, 0)),
    ],
    out_shape=[
        jax.ShapeDtypeStruct((N, D), jnp.float32),
        jax.ShapeDtypeStruct((N, 16), jnp.float32),
        jax.ShapeDtypeStruct((1, 16), jnp.float32),
    ],
)

_mid_call = pl.pallas_call(
    _tc_mid,
    grid=(N // BR,),
    in_specs=[
        pl.BlockSpec((1, BR, ROW1), lambda i: (0, i, 0)),
        pl.BlockSpec((1, BR, ROW1), lambda i: (1, i, 0)),
        pl.BlockSpec((H, HC1), lambda i: (0, 0)),
        pl.BlockSpec((1, HC1), lambda i: (0, 0)),
        pl.BlockSpec((HC1, HC2), lambda i: (0, 0)),
        pl.BlockSpec((HC2, 16), lambda i: (0, 0)),
    ],
    out_specs=[
        pl.BlockSpec((BR, HC2), lambda i: (i, 0)),
        pl.BlockSpec((BR, 16), lambda i: (i, 0)),
        pl.BlockSpec((1, 16), lambda i: (0, 0)),
    ],
    out_shape=[
        jax.ShapeDtypeStruct((N, HC2), jnp.float32),
        jax.ShapeDtypeStruct((N, 16), jnp.float32),
        jax.ShapeDtypeStruct((1, 16), jnp.float32),
    ],
)

_out_call = pl.pallas_call(
    _tc_out,
    grid=(N // BR,),
    in_specs=[
        pl.BlockSpec((1, BR, ROW2), lambda i: (0, i, 0)),
        pl.BlockSpec((1, BR, ROW2), lambda i: (1, i, 0)),
        pl.BlockSpec((H, HC2), lambda i: (0, 0)),
        pl.BlockSpec((1, HC2), lambda i: (0, 0)),
    ],
    out_specs=pl.BlockSpec((BR, HC2), lambda i: (i, 0)),
    out_shape=jax.ShapeDtypeStruct((N, HC2), jnp.float32),
)


def _asel(a_s, a_d, c):
    eye = jnp.eye(H, dtype=jnp.float32)
    s = (a_s[:, :, None] * eye[:, None, :]).reshape(H * c, H)
    d = (a_d[:, :, None] * eye[:, None, :]).reshape(H * c, H)
    return jnp.concatenate([s, d], axis=1)


def _mtile(m):
    mm = m[0, :8] + m[0, 8:]
    mm = jnp.where(mm >= 0.0, mm, 0.2 * mm)
    return jnp.concatenate([mm, mm])


def kernel(x, edge_index, W1, a1_src, a1_dst, b1, W2, a2_src, a2_dst, b2):
    loop = jnp.arange(N, dtype=jnp.int32)
    npad = ET_PAD - ET
    src = jnp.concatenate([edge_index[0], loop, jnp.zeros((npad,), jnp.int32)])
    dst = jnp.concatenate([edge_index[1], loop, jnp.full((npad,), N, jnp.int32)])

    asel1 = _asel(a1_src, a1_dst, C1)
    asel2 = _asel(a2_src, a2_dst, C2)
    rsel1 = jnp.repeat(jnp.eye(H, dtype=jnp.float32), C1, axis=1)
    rsel2 = jnp.repeat(jnp.eye(H, dtype=jnp.float32), C2, axis=1)

    h1, al1, m1 = _embed_call(x, W1, asel1)
    part1 = _sc_gat1(src, dst, al1, h1, _mtile(m1))
    h2, al2, m2 = _mid_call(part1, part1, rsel1, b1.reshape(1, HC1), W2, asel2)
    part2 = _sc_gat2(src, dst, al2, h2, _mtile(m2))
    return _out_call(part2, part2, rsel2, b2.reshape(1, HC2))


# speedup vs baseline: 1.6245x; 1.0500x over previous
import jax
import jax.numpy as jnp
from jax import lax
from jax.experimental import pallas as pl
from jax.experimental.pallas import tpu as pltpu
from jax.experimental.pallas import tpu_sc as plsc

N = 10000
E = 320000
D = 128
H = 8
C1, C2 = 16, 8
HC1, HC2 = H * C1, H * C2
ROW1, ROW2 = HC1 + 16, HC2 + 16
NPAD = 10016
NC, NS = 2, 16
K = 64
PT = 10368
NCH = PT // K
ET = E + N
ET_PAD = NC * NS * PT
RPT = NPAD // NS
BR = 1000


def _make_sc_gat(hc, row):
    jb = hc // 16

    def body(src_hbm, dst_hbm, alpha_hbm, h_hbm, m_hbm, out_hbm,
             table, srcv0, srcv1, dstv0, dstv1, sdst0, sdst1, dprime,
             av0, av1, bv0, bv1, hb0, hb1, mg0, mg1, wbuf, mbuf,
             gsem0, gsem1, isem0, isem1, ssem0, ssem1):
        srcv, dstv, sdst = [srcv0, srcv1], [dstv0, dstv1], [sdst0, sdst1]
        av, bv, hbuf, msgc = [av0, av1], [bv0, bv1], [hb0, hb1], [mg0, mg1]
        gsem, isem, ssem = [gsem0, gsem1], [isem0, isem1], [ssem0, ssem1]
        cid = lax.axis_index("c")
        sid = lax.axis_index("s")
        tb = (cid * NS + sid) * PT
        lane = lax.iota(jnp.int32, 16)
        half = lane // 8
        hcol = lane - 8 * half
        zero16 = jnp.zeros((16,), jnp.float32)
        izero16 = jnp.zeros((16,), jnp.int32)

        pltpu.sync_copy(m_hbm, mbuf)

        def zbody(i, c):
            iv = jnp.full((16,), i, jnp.int32)
            for j in range(row // 16):
                plsc.store_scatter(msgc[0], [iv, lane + 16 * j], zero16)
                plsc.store_scatter(msgc[1], [iv, lane + 16 * j], zero16)
            return c
        lax.fori_loop(0, K, zbody, 0, unroll=4)
        for i in range(K // 16):
            dprime[pl.ds(16 * i, 16)] = izero16
        nfull = RPT // K
        for b in range(nfull):
            pltpu.sync_copy(msgc[0], table.at[pl.ds(sid * RPT + b * K, K)])
        rem = RPT - nfull * K
        if rem:
            pltpu.sync_copy(msgc[0].at[pl.ds(0, rem)],
                            table.at[pl.ds(sid * RPT + nfull * K, rem)])
        plsc.subcore_barrier()

        def issue_idx(p, gi):
            base = tb + gi * K
            pltpu.async_copy(src_hbm.at[pl.ds(base, K)], srcv[p], isem[p])
            pltpu.async_copy(dst_hbm.at[pl.ds(base, K)], dstv[p], isem[p])

        def wait_idx(p):
            pltpu.make_async_copy(src_hbm.at[pl.ds(0, K)], srcv[p], isem[p]).wait()
            pltpu.make_async_copy(dst_hbm.at[pl.ds(0, K)], dstv[p], isem[p]).wait()

        def issue_g(p):
            pltpu.async_copy(alpha_hbm.at[srcv[p]], av[p], gsem[p])
            pltpu.async_copy(alpha_hbm.at[dstv[p]], bv[p], gsem[p])
            pltpu.async_copy(h_hbm.at[srcv[p]], hbuf[p], gsem[p])

        def wait_g(p):
            pltpu.make_async_copy(alpha_hbm.at[srcv[p]], av[p], gsem[p]).wait()
            pltpu.make_async_copy(alpha_hbm.at[dstv[p]], bv[p], gsem[p]).wait()
            pltpu.make_async_copy(h_hbm.at[srcv[p]], hbuf[p], gsem[p]).wait()

        def drain_s(p):
            pltpu.make_async_copy(msgc[p], table.at[dprime], ssem[p]).wait()

        pltpu.async_copy(msgc[0], table.at[dprime], ssem[0], add=True)
        pltpu.async_copy(msgc[1], table.at[dprime], ssem[1], add=True)
        issue_idx(0, 0)
        wait_idx(0)
        issue_g(0)
        issue_idx(1, 1)

        def phase(p, g):
            wait_g(p)
            drain_s(p)
            for i in range(K // 16):
                sdst[p][pl.ds(16 * i, 16)] = dstv[p][pl.ds(16 * i, 16)]
            wait_idx(1 - p)
            issue_g(1 - p)
            issue_idx(p, jnp.minimum(g + 2, NCH - 1))
            mv = mbuf[...]

            def wbody(v, cc):
                rows = 2 * v + half
                xs = plsc.load_gather(av[p], [rows, hcol])
                xd = plsc.load_gather(bv[p], [rows, hcol + 8])
                e = xs + xd
                e = jnp.where(e >= 0.0, e, 0.2 * e)
                w = jnp.exp(e - mv)
                wbuf[pl.ds(v * 16, 16)] = w
                plsc.store_scatter(msgc[p], [rows, hcol + hc], w)
                plsc.store_scatter(msgc[p], [rows, hcol + hc + 8], w)
                return cc
            lax.fori_loop(0, K // 2, wbody, 0, unroll=4)

            def mbody(e2, cc):
                for j in range(jb):
                    if hc == 128:
                        widx = jnp.full((16,), e2 * 8 + j, jnp.int32)
                    else:
                        widx = 8 * e2 + 2 * j + half
                    ws = plsc.load_gather(wbuf, [widx])
                    hv = hbuf[p][e2, pl.ds(16 * j, 16)]
                    msgc[p][e2, pl.ds(16 * j, 16)] = hv * ws
                return cc
            lax.fori_loop(0, K, mbody, 0, unroll=2)

            pltpu.async_copy(msgc[p], table.at[sdst[p]], ssem[p], add=True)

        def loop(gg, c):
            phase(0, 2 * gg)
            phase(1, 2 * gg + 1)
            return c
        lax.fori_loop(0, NCH // 2, loop, 0)

        wait_g(0)
        wait_idx(1)
        drain_s(0)
        drain_s(1)

        plsc.subcore_barrier()
        for b in range(2):
            r0 = sid * RPT + b * (RPT // 2)
            pltpu.sync_copy(table.at[pl.ds(r0, RPT // 2)],
                            out_hbm.at[cid, pl.ds(r0, RPT // 2)])

    return body


def _sc_call(hc, row):
    mesh = plsc.VectorSubcoreMesh(
        core_axis_name="c", subcore_axis_name="s", num_cores=NC, num_subcores=NS)
    return pl.kernel(
        _make_sc_gat(hc, row),
        out_type=jax.ShapeDtypeStruct((NC, NPAD, row), jnp.float32),
        mesh=mesh,
        compiler_params=pltpu.CompilerParams(
            needs_layout_passes=False, use_tc_tiling_on_sc=False),
        scratch_types=(
            [pltpu.VMEM_SHARED((NPAD, row), jnp.float32)]
            + [pltpu.VMEM((K,), jnp.int32) for _ in range(7)]
            + [pltpu.VMEM((K, 16), jnp.float32) for _ in range(4)]
            + [pltpu.VMEM((K, hc), jnp.float32) for _ in range(2)]
            + [pltpu.VMEM((K, row), jnp.float32) for _ in range(2)]
            + [pltpu.VMEM((K * 8,), jnp.float32),
               pltpu.VMEM((16,), jnp.float32)]
            + [pltpu.SemaphoreType.DMA for _ in range(6)]
        ),
    )


_sc_gat1 = _sc_call(HC1, ROW1)
_sc_gat2 = _sc_call(HC2, ROW2)


def _tc_embed(x_ref, w_ref, asel_ref, h_ref, a_ref, m_ref):
    i = pl.program_id(0)
    hb = jnp.dot(x_ref[...], w_ref[...], preferred_element_type=jnp.float32)
    ab = jnp.dot(hb, asel_ref[...], preferred_element_type=jnp.float32)
    h_ref[...] = hb
    a_ref[...] = ab
    bm = jnp.max(ab, axis=0, keepdims=True)

    @pl.when(i == 0)
    def _init():
        m_ref[...] = bm

    @pl.when(i != 0)
    def _acc():
        m_ref[...] = jnp.maximum(m_ref[...], bm)


def _tc_mid(p0_ref, p1_ref, rsel_ref, b1_ref, w2_ref, asel_ref,
            h_ref, a_ref, m_ref):
    i = pl.program_id(0)
    p = p0_ref[0] + p1_ref[0]
    num = p[:, :HC1]
    den = p[:, HC1:HC1 + 8]
    rec = 1.0 / (den + 1e-30)
    o1 = num * jnp.dot(rec, rsel_ref[...], preferred_element_type=jnp.float32)
    o1 = jnp.maximum(o1 + b1_ref[...], 0.0)
    h2 = jnp.dot(o1, w2_ref[...], preferred_element_type=jnp.float32)
    a2 = jnp.dot(h2, asel_ref[...], preferred_element_type=jnp.float32)
    h_ref[...] = h2
    a_ref[...] = a2
    bm = jnp.max(a2, axis=0, keepdims=True)

    @pl.when(i == 0)
    def _init():
        m_ref[...] = bm

    @pl.when(i != 0)
    def _acc():
        m_ref[...] = jnp.maximum(m_ref[...], bm)


def _tc_out(p0_ref, p1_ref, rsel_ref, b2_ref, o_ref):
    p = p0_ref[0] + p1_ref[0]
    num = p[:, :HC2]
    den = p[:, HC2:HC2 + 8]
    rec = 1.0 / (den + 1e-30)
    o = num * jnp.dot(rec, rsel_ref[...], preferred_element_type=jnp.float32)
    o = o + b2_ref[...]
    m = jnp.max(o, axis=1, keepdims=True)
    l = o - m
    o_ref[...] = l - jnp.log(jnp.sum(jnp.exp(l), axis=1, keepdims=True))


_embed_call = pl.pallas_call(
    _tc_embed,
    grid=(N // BR,),
    in_specs=[
        pl.BlockSpec((BR, D), lambda i: (i, 0)),
        pl.BlockSpec((D, D), lambda i: (0, 0)),
        pl.BlockSpec((D, 16), lambda i: (0, 0)),
    ],
    out_specs=[
        pl.BlockSpec((BR, D), lambda i: (i, 0)),
        pl.BlockSpec((BR, 16), lambda i: (i, 0)),
        pl.BlockSpec((1, 16), lambda i: (0, 0)),
    ],
    out_shape=[
        jax.ShapeDtypeStruct((N, D), jnp.float32),
        jax.ShapeDtypeStruct((N, 16), jnp.float32),
        jax.ShapeDtypeStruct((1, 16), jnp.float32),
    ],
)

_mid_call = pl.pallas_call(
    _tc_mid,
    grid=(N // BR,),
    in_specs=[
        pl.BlockSpec((1, BR, ROW1), lambda i: (0, i, 0)),
        pl.BlockSpec((1, BR, ROW1), lambda i: (1, i, 0)),
        pl.BlockSpec((H, HC1), lambda i: (0, 0)),
        pl.BlockSpec((1, HC1), lambda i: (0, 0)),
        pl.BlockSpec((HC1, HC2), lambda i: (0, 0)),
        pl.BlockSpec((HC2, 16), lambda i: (0, 0)),
    ],
    out_specs=[
        pl.BlockSpec((BR, HC2), lambda i: (i, 0)),
        pl.BlockSpec((BR, 16), lambda i: (i, 0)),
        pl.BlockSpec((1, 16), lambda i: (0, 0)),
    ],
    out_shape=[
        jax.ShapeDtypeStruct((N, HC2), jnp.float32),
        jax.ShapeDtypeStruct((N, 16), jnp.float32),
        jax.ShapeDtypeStruct((1, 16), jnp.float32),
    ],
)

_out_call = pl.pallas_call(
    _tc_out,
    grid=(N // BR,),
    in_specs=[
        pl.BlockSpec((1, BR, ROW2), lambda i: (0, i, 0)),
        pl.BlockSpec((1, BR, ROW2), lambda i: (1, i, 0)),
        pl.BlockSpec((H, HC2), lambda i: (0, 0)),
        pl.BlockSpec((1, HC2), lambda i: (0, 0)),
    ],
    out_specs=pl.BlockSpec((BR, HC2), lambda i: (i, 0)),
    out_shape=jax.ShapeDtypeStruct((N, HC2), jnp.float32),
)


def _asel(a_s, a_d, c):
    eye = jnp.eye(H, dtype=jnp.float32)
    s = (a_s[:, :, None] * eye[:, None, :]).reshape(H * c, H)
    d = (a_d[:, :, None] * eye[:, None, :]).reshape(H * c, H)
    return jnp.concatenate([s, d], axis=1)


def _mtile(m):
    mm = m[0, :8] + m[0, 8:]
    mm = jnp.where(mm >= 0.0, mm, 0.2 * mm)
    return jnp.concatenate([mm, mm])


def kernel(x, edge_index, W1, a1_src, a1_dst, b1, W2, a2_src, a2_dst, b2):
    loop = jnp.arange(N, dtype=jnp.int32)
    npad = ET_PAD - ET
    src = jnp.concatenate([edge_index[0], loop, jnp.zeros((npad,), jnp.int32)])
    dst = jnp.concatenate([edge_index[1], loop, jnp.full((npad,), N, jnp.int32)])

    asel1 = _asel(a1_src, a1_dst, C1)
    asel2 = _asel(a2_src, a2_dst, C2)
    rsel1 = jnp.repeat(jnp.eye(H, dtype=jnp.float32), C1, axis=1)
    rsel2 = jnp.repeat(jnp.eye(H, dtype=jnp.float32), C2, axis=1)

    h1, al1, m1 = _embed_call(x, W1, asel1)
    part1 = _sc_gat1(src, dst, al1, h1, _mtile(m1))
    h2, al2, m2 = _mid_call(part1, part1, rsel1, b1.reshape(1, HC1), W2, asel2)
    part2 = _sc_gat2(src, dst, al2, h2, _mtile(m2))
    return _out_call(part2, part2, rsel2, b2.reshape(1, HC2))


# speedup vs baseline: 2.6938x; 1.6582x over previous
import jax
import jax.numpy as jnp
from jax import lax
from jax.experimental import pallas as pl
from jax.experimental.pallas import tpu as pltpu
from jax.experimental.pallas import tpu_sc as plsc

N = 10000
E = 320000
D = 128
H = 8
C1, C2 = 16, 8
HC1, HC2 = H * C1, H * C2
ROW1, ROW2 = HC1 + 16, HC2 + 16
NPAD = 10016
NC, NS = 2, 16
K = 64
PT = 10368
NCH = PT // K
ET = E + N
ET_PAD = NC * NS * PT
RPT = NPAD // NS
BR = 1000


def _make_sc_gat(hc, row):
    jb = hc // 16

    def body(src_hbm, dst_hbm, alpha_hbm, h_hbm, m_hbm, out_hbm,
             table, srcv0, srcv1, dstv0, dstv1, sdst0, sdst1, dprime,
             av0, av1, bv0, bv1, hb0, hb1, mg0, mg1, wbuf, mbuf,
             gsem0, gsem1, isem0, isem1, ssem0, ssem1):
        srcv, dstv, sdst = [srcv0, srcv1], [dstv0, dstv1], [sdst0, sdst1]
        av, bv, hbuf, msgc = [av0, av1], [bv0, bv1], [hb0, hb1], [mg0, mg1]
        gsem, isem, ssem = [gsem0, gsem1], [isem0, isem1], [ssem0, ssem1]
        cid = lax.axis_index("c")
        sid = lax.axis_index("s")
        tb = (cid * NS + sid) * PT
        lane = lax.iota(jnp.int32, 16)
        half = lane // 8
        hcol = lane - 8 * half
        zero16 = jnp.zeros((16,), jnp.float32)
        izero16 = jnp.zeros((16,), jnp.int32)

        pltpu.sync_copy(m_hbm, mbuf)

        @plsc.parallel_loop(0, K, 1, unroll=4)
        def zbody(i):
            for j in range(row // 16):
                msgc[0][i, pl.ds(16 * j, 16)] = zero16
                msgc[1][i, pl.ds(16 * j, 16)] = zero16
        for i in range(K // 16):
            dprime[pl.ds(16 * i, 16)] = izero16
        nfull = RPT // K
        for b in range(nfull):
            pltpu.sync_copy(msgc[0], table.at[pl.ds(sid * RPT + b * K, K)])
        rem = RPT - nfull * K
        if rem:
            pltpu.sync_copy(msgc[0].at[pl.ds(0, rem)],
                            table.at[pl.ds(sid * RPT + nfull * K, rem)])
        plsc.subcore_barrier()

        def issue_idx(p, gi):
            base = tb + gi * K
            pltpu.async_copy(src_hbm.at[pl.ds(base, K)], srcv[p], isem[p])
            pltpu.async_copy(dst_hbm.at[pl.ds(base, K)], dstv[p], isem[p])

        def wait_idx(p):
            pltpu.make_async_copy(src_hbm.at[pl.ds(0, K)], srcv[p], isem[p]).wait()
            pltpu.make_async_copy(dst_hbm.at[pl.ds(0, K)], dstv[p], isem[p]).wait()

        def issue_g(p):
            pltpu.async_copy(alpha_hbm.at[srcv[p]], av[p], gsem[p])
            pltpu.async_copy(alpha_hbm.at[dstv[p]], bv[p], gsem[p])
            pltpu.async_copy(h_hbm.at[srcv[p]], hbuf[p], gsem[p])

        def wait_g(p):
            pltpu.make_async_copy(alpha_hbm.at[srcv[p]], av[p], gsem[p]).wait()
            pltpu.make_async_copy(alpha_hbm.at[dstv[p]], bv[p], gsem[p]).wait()
            pltpu.make_async_copy(h_hbm.at[srcv[p]], hbuf[p], gsem[p]).wait()

        def drain_s(p):
            pltpu.make_async_copy(msgc[p], table.at[dprime], ssem[p]).wait()

        pltpu.async_copy(msgc[0], table.at[dprime], ssem[0], add=True)
        pltpu.async_copy(msgc[1], table.at[dprime], ssem[1], add=True)
        issue_idx(0, 0)
        wait_idx(0)
        issue_g(0)
        issue_idx(1, 1)

        def phase(p, g):
            wait_g(p)
            drain_s(p)
            for i in range(K // 16):
                sdst[p][pl.ds(16 * i, 16)] = dstv[p][pl.ds(16 * i, 16)]
            wait_idx(1 - p)
            issue_g(1 - p)
            issue_idx(p, jnp.minimum(g + 2, NCH - 1))
            mv = mbuf[...]

            @plsc.parallel_loop(0, K // 2, 1, unroll=4)
            def wbody(v):
                rows = 2 * v + half
                xs = plsc.load_gather(av[p], [rows, hcol])
                xd = plsc.load_gather(bv[p], [rows, hcol + 8])
                e = xs + xd
                e = jnp.where(e >= 0.0, e, 0.2 * e)
                w = jnp.exp(e - mv)
                wbuf[pl.ds(v * 16, 16)] = w
                plsc.store_scatter(msgc[p], [rows, hcol + hc], w)
                plsc.store_scatter(msgc[p], [rows, hcol + hc + 8], w)

            @plsc.parallel_loop(0, K, 1, unroll=2)
            def mbody(e2):
                for j in range(jb):
                    if hc == 128:
                        widx = jnp.full((16,), e2 * 8 + j, jnp.int32)
                    else:
                        widx = 8 * e2 + 2 * j + half
                    ws = plsc.load_gather(wbuf, [widx])
                    hv = hbuf[p][e2, pl.ds(16 * j, 16)]
                    msgc[p][e2, pl.ds(16 * j, 16)] = hv * ws

            pltpu.async_copy(msgc[p], table.at[sdst[p]], ssem[p], add=True)

        def loop(gg, c):
            phase(0, 2 * gg)
            phase(1, 2 * gg + 1)
            return c
        lax.fori_loop(0, NCH // 2, loop, 0)

        wait_g(0)
        wait_idx(1)
        drain_s(0)
        drain_s(1)

        plsc.subcore_barrier()
        for b in range(2):
            r0 = sid * RPT + b * (RPT // 2)
            pltpu.sync_copy(table.at[pl.ds(r0, RPT // 2)],
                            out_hbm.at[cid, pl.ds(r0, RPT // 2)])

    return body


def _sc_call(hc, row):
    mesh = plsc.VectorSubcoreMesh(
        core_axis_name="c", subcore_axis_name="s", num_cores=NC, num_subcores=NS)
    return pl.kernel(
        _make_sc_gat(hc, row),
        out_type=jax.ShapeDtypeStruct((NC, NPAD, row), jnp.float32),
        mesh=mesh,
        compiler_params=pltpu.CompilerParams(
            needs_layout_passes=False, use_tc_tiling_on_sc=False),
        scratch_types=(
            [pltpu.VMEM_SHARED((NPAD, row), jnp.float32)]
            + [pltpu.VMEM((K,), jnp.int32) for _ in range(7)]
            + [pltpu.VMEM((K, 16), jnp.float32) for _ in range(4)]
            + [pltpu.VMEM((K, hc), jnp.float32) for _ in range(2)]
            + [pltpu.VMEM((K, row), jnp.float32) for _ in range(2)]
            + [pltpu.VMEM((K * 8,), jnp.float32),
               pltpu.VMEM((16,), jnp.float32)]
            + [pltpu.SemaphoreType.DMA for _ in range(6)]
        ),
    )


_sc_gat1 = _sc_call(HC1, ROW1)
_sc_gat2 = _sc_call(HC2, ROW2)


def _tc_embed(x_ref, w_ref, asel_ref, h_ref, a_ref, m_ref):
    i = pl.program_id(0)
    hb = jnp.dot(x_ref[...], w_ref[...], preferred_element_type=jnp.float32)
    ab = jnp.dot(hb, asel_ref[...], preferred_element_type=jnp.float32)
    h_ref[...] = hb
    a_ref[...] = ab
    bm = jnp.max(ab, axis=0, keepdims=True)

    @pl.when(i == 0)
    def _init():
        m_ref[...] = bm

    @pl.when(i != 0)
    def _acc():
        m_ref[...] = jnp.maximum(m_ref[...], bm)


def _tc_mid(p0_ref, p1_ref, rsel_ref, b1_ref, w2_ref, asel_ref,
            h_ref, a_ref, m_ref):
    i = pl.program_id(0)
    p = p0_ref[0] + p1_ref[0]
    num = p[:, :HC1]
    den = p[:, HC1:HC1 + 8]
    rec = 1.0 / (den + 1e-30)
    o1 = num * jnp.dot(rec, rsel_ref[...], preferred_element_type=jnp.float32)
    o1 = jnp.maximum(o1 + b1_ref[...], 0.0)
    h2 = jnp.dot(o1, w2_ref[...], preferred_element_type=jnp.float32)
    a2 = jnp.dot(h2, asel_ref[...], preferred_element_type=jnp.float32)
    h_ref[...] = h2
    a_ref[...] = a2
    bm = jnp.max(a2, axis=0, keepdims=True)

    @pl.when(i == 0)
    def _init():
        m_ref[...] = bm

    @pl.when(i != 0)
    def _acc():
        m_ref[...] = jnp.maximum(m_ref[...], bm)


def _tc_out(p0_ref, p1_ref, rsel_ref, b2_ref, o_ref):
    p = p0_ref[0] + p1_ref[0]
    num = p[:, :HC2]
    den = p[:, HC2:HC2 + 8]
    rec = 1.0 / (den + 1e-30)
    o = num * jnp.dot(rec, rsel_ref[...], preferred_element_type=jnp.float32)
    o = o + b2_ref[...]
    m = jnp.max(o, axis=1, keepdims=True)
    l = o - m
    o_ref[...] = l - jnp.log(jnp.sum(jnp.exp(l), axis=1, keepdims=True))


_embed_call = pl.pallas_call(
    _tc_embed,
    grid=(N // BR,),
    in_specs=[
        pl.BlockSpec((BR, D), lambda i: (i, 0)),
        pl.BlockSpec((D, D), lambda i: (0, 0)),
        pl.BlockSpec((D, 16), lambda i: (0, 0)),
    ],
    out_specs=[
        pl.BlockSpec((BR, D), lambda i: (i, 0)),
        pl.BlockSpec((BR, 16), lambda i: (i, 0)),
        pl.BlockSpec((1, 16), lambda i: (0, 0)),
    ],
    out_shape=[
        jax.ShapeDtypeStruct((N, D), jnp.float32),
        jax.ShapeDtypeStruct((N, 16), jnp.float32),
        jax.ShapeDtypeStruct((1, 16), jnp.float32),
    ],
)

_mid_call = pl.pallas_call(
    _tc_mid,
    grid=(N // BR,),
    in_specs=[
        pl.BlockSpec((1, BR, ROW1), lambda i: (0, i, 0)),
        pl.BlockSpec((1, BR, ROW1), lambda i: (1, i, 0)),
        pl.BlockSpec((H, HC1), lambda i: (0, 0)),
        pl.BlockSpec((1, HC1), lambda i: (0, 0)),
        pl.BlockSpec((HC1, HC2), lambda i: (0, 0)),
        pl.BlockSpec((HC2, 16), lambda i: (0, 0)),
    ],
    out_specs=[
        pl.BlockSpec((BR, HC2), lambda i: (i, 0)),
        pl.BlockSpec((BR, 16), lambda i: (i, 0)),
        pl.BlockSpec((1, 16), lambda i: (0, 0)),
    ],
    out_shape=[
        jax.ShapeDtypeStruct((N, HC2), jnp.float32),
        jax.ShapeDtypeStruct((N, 16), jnp.float32),
        jax.ShapeDtypeStruct((1, 16), jnp.float32),
    ],
)

_out_call = pl.pallas_call(
    _tc_out,
    grid=(N // BR,),
    in_specs=[
        pl.BlockSpec((1, BR, ROW2), lambda i: (0, i, 0)),
        pl.BlockSpec((1, BR, ROW2), lambda i: (1, i, 0)),
        pl.BlockSpec((H, HC2), lambda i: (0, 0)),
        pl.BlockSpec((1, HC2), lambda i: (0, 0)),
    ],
    out_specs=pl.BlockSpec((BR, HC2), lambda i: (i, 0)),
    out_shape=jax.ShapeDtypeStruct((N, HC2), jnp.float32),
)


def _asel(a_s, a_d, c):
    eye = jnp.eye(H, dtype=jnp.float32)
    s = (a_s[:, :, None] * eye[:, None, :]).reshape(H * c, H)
    d = (a_d[:, :, None] * eye[:, None, :]).reshape(H * c, H)
    return jnp.concatenate([s, d], axis=1)


def _mtile(m):
    mm = m[0, :8] + m[0, 8:]
    mm = jnp.where(mm >= 0.0, mm, 0.2 * mm)
    return jnp.concatenate([mm, mm])


def kernel(x, edge_index, W1, a1_src, a1_dst, b1, W2, a2_src, a2_dst, b2):
    loop = jnp.arange(N, dtype=jnp.int32)
    npad = ET_PAD - ET
    src = jnp.concatenate([edge_index[0], loop, jnp.zeros((npad,), jnp.int32)])
    dst = jnp.concatenate([edge_index[1], loop, jnp.full((npad,), N, jnp.int32)])

    asel1 = _asel(a1_src, a1_dst, C1)
    asel2 = _asel(a2_src, a2_dst, C2)
    rsel1 = jnp.repeat(jnp.eye(H, dtype=jnp.float32), C1, axis=1)
    rsel2 = jnp.repeat(jnp.eye(H, dtype=jnp.float32), C2, axis=1)

    h1, al1, m1 = _embed_call(x, W1, asel1)
    part1 = _sc_gat1(src, dst, al1, h1, _mtile(m1))
    h2, al2, m2 = _mid_call(part1, part1, rsel1, b1.reshape(1, HC1), W2, asel2)
    part2 = _sc_gat2(src, dst, al2, h2, _mtile(m2))
    return _out_call(part2, part2, rsel2, b2.reshape(1, HC2))
